# baseline (device time: 59261 ns/iter reference)
import jax
import jax.numpy as jnp
from jax import lax
from jax.experimental import pallas as pl
from jax.experimental.pallas import tpu as pltpu


def kernel(x, pi):
    my = lax.axis_index("i")
    dst = lax.dynamic_index_in_dim(pi, my, keepdims=False).astype(jnp.int32)
    src = jnp.argmax(pi == my).astype(jnp.int32)
    meta = jnp.stack([dst, src])

    def body(meta_ref, x_ref, out_ref, send_sem, recv_sem):
        dst_id = meta_ref[0]
        src_id = meta_ref[1]

        barrier = pltpu.get_barrier_semaphore()
        pl.semaphore_signal(
            barrier,
            inc=1,
            device_id=(src_id,),
            device_id_type=pl.DeviceIdType.MESH,
        )
        pl.semaphore_wait(barrier, 1)

        rdma = pltpu.make_async_remote_copy(
            src_ref=x_ref,
            dst_ref=out_ref,
            send_sem=send_sem,
            recv_sem=recv_sem,
            device_id=(dst_id,),
            device_id_type=pl.DeviceIdType.MESH,
        )
        rdma.start()
        rdma.wait_send()
        rdma.wait_recv()

    return pl.pallas_call(
        body,
        out_shape=jax.ShapeDtypeStruct(x.shape, x.dtype),
        in_specs=[
            pl.BlockSpec(memory_space=pltpu.SMEM),
            pl.BlockSpec(memory_space=pltpu.VMEM),
        ],
        out_specs=pl.BlockSpec(memory_space=pltpu.VMEM),
        scratch_shapes=[
            pltpu.SemaphoreType.DMA,
            pltpu.SemaphoreType.DMA,
        ],
        compiler_params=pltpu.CompilerParams(collective_id=0),
    )(meta, x)
